# Initial kernel scaffold; baseline (speedup 1.0000x reference)
import jax
import jax.numpy as jnp
from jax.experimental import pallas as pl


def kernel(idx, token_emb, pos_emb):
    raise NotImplementedError("write your pallas kernel here")


# speedup vs baseline: 2.6937x; 2.6937x over previous
import functools

import jax
import jax.numpy as jnp
from jax import lax
from jax.experimental import pallas as pl
from jax.experimental.pallas import tpu as pltpu
from jax.experimental.pallas import tpu_sc as plsc

NC = 2
NS = 16
NW = NC * NS
LANES = 16
S0 = 104
IDXPAD = 128


def _make_sc_lookup(V, D, B, T):
    s1 = T - S0
    assert 0 < s1 <= IDXPAD and s1 % 8 == 0 and T % 8 == 0
    assert B % NW == 0
    cpw = B // NW
    assert cpw % 2 == 0

    mesh = plsc.VectorSubcoreMesh(core_axis_name="c", subcore_axis_name="s")

    @functools.partial(
        pl.kernel,
        out_type=jax.ShapeDtypeStruct((B * T, D), jnp.float32),
        mesh=mesh,
        scratch_types=[
            pltpu.VMEM((cpw, 2, IDXPAD), jnp.int32),
            pltpu.VMEM((T, D), jnp.float32),
            pltpu.VMEM((T, D), jnp.float32),
            pltpu.VMEM((T, D), jnp.float32),
            pltpu.SemaphoreType.DMA,
            pltpu.SemaphoreType.DMA,
            pltpu.SemaphoreType.DMA,
            pltpu.SemaphoreType.DMA,
        ],
    )
    def lookup(tok_hbm, idx_hbm, pos_hbm, out_hbm,
               idx_v, pos_v, rows0, rows1, g0, g1, s0, s1_):
        wid = lax.axis_index("s") * NC + lax.axis_index("c")
        seq0 = wid * cpw

        pltpu.sync_copy(pos_hbm.at[pl.ds(0, T)], pos_v)
        pltpu.sync_copy(idx_hbm.at[pl.ds(seq0, cpw)], idx_v)

        rows = (rows0, rows1)
        gsem = (g0, g1)
        ssem = (s0, s1_)

        def gather_start(c, buf):
            pltpu.async_copy(
                tok_hbm.at[idx_v.at[c, 0, pl.ds(0, S0)]],
                rows[buf].at[pl.ds(0, S0)], gsem[buf])
            pltpu.async_copy(
                tok_hbm.at[idx_v.at[c, 1, pl.ds(0, s1)]],
                rows[buf].at[pl.ds(S0, s1)], gsem[buf])

        def gather_wait(buf):
            pltpu.make_async_copy(
                tok_hbm.at[idx_v.at[0, 0, pl.ds(0, S0)]],
                rows[buf].at[pl.ds(0, S0)], gsem[buf]).wait()
            pltpu.make_async_copy(
                tok_hbm.at[idx_v.at[0, 1, pl.ds(0, s1)]],
                rows[buf].at[pl.ds(S0, s1)], gsem[buf]).wait()

        def scatter_start(c, buf):
            pltpu.async_copy(
                rows[buf], out_hbm.at[pl.ds((seq0 + c) * T, T)], ssem[buf])

        def scatter_wait(buf):
            pltpu.make_async_copy(
                rows[buf], out_hbm.at[pl.ds(0, T)], ssem[buf]).wait()

        def add_pos(buf):
            def row_body(r, _):
                for q in range(D // LANES):
                    sl = pl.ds(q * LANES, LANES)
                    rows[buf][r, sl] += pos_v[r, sl]
                return 0
            lax.fori_loop(0, T, row_body, 0, unroll=2)

        gather_start(0, 0)

        def outer(c2, _):
            for par in range(2):
                c = c2 * 2 + par
                buf = par
                nbuf = 1 - par

                @pl.when(c + 1 < cpw)
                def _start_next():
                    @pl.when(c >= 1)
                    def _drain():
                        scatter_wait(nbuf)
                    gather_start(c + 1, nbuf)

                gather_wait(buf)
                add_pos(buf)
                scatter_start(c, buf)
            return 0

        lax.fori_loop(0, cpw // 2, outer, 0)
        scatter_wait(0)
        scatter_wait(1)

    return lookup


def kernel(idx, token_emb, pos_emb):
    B, T = idx.shape
    V, D = token_emb.shape
    idx = idx.astype(jnp.int32)
    h0 = jnp.pad(idx[:, :S0], ((0, 0), (0, IDXPAD - S0)))
    h1 = jnp.pad(idx[:, S0:], ((0, 0), (0, IDXPAD - (T - S0))))
    idx3 = jnp.stack([h0, h1], axis=1)
    lookup = _make_sc_lookup(V, D, B, T)
    out = lookup(token_emb, idx3, pos_emb)
    return out.reshape(B, T, D)


# speedup vs baseline: 8.7396x; 3.2445x over previous
import functools

import jax
import jax.numpy as jnp
from jax import lax
from jax.experimental import pallas as pl
from jax.experimental.pallas import tpu as pltpu
from jax.experimental.pallas import tpu_sc as plsc

NC = 2
NS = 16
NW = NC * NS
LANES = 16
S0 = 104
IDXPAD = 128
NBUF = 4


def _make_sc_lookup(V, D, B, T):
    s1 = T - S0
    assert 0 < s1 <= IDXPAD and s1 % 8 == 0 and T % 8 == 0
    assert B % NW == 0
    cpw = B // NW
    assert cpw % NBUF == 0 and cpw >= 2 * NBUF

    mesh = plsc.VectorSubcoreMesh(core_axis_name="c", subcore_axis_name="s")

    @functools.partial(
        pl.kernel,
        out_type=jax.ShapeDtypeStruct((B * T, D), jnp.float32),
        mesh=mesh,
        scratch_types=[
            pltpu.VMEM((2, 2, IDXPAD), jnp.int32),
            pltpu.VMEM((T, D), jnp.float32),
            [pltpu.VMEM((T, D), jnp.float32) for _ in range(NBUF)],
            [pltpu.SemaphoreType.DMA for _ in range(2)],
            [pltpu.SemaphoreType.DMA for _ in range(NBUF)],
            [pltpu.SemaphoreType.DMA for _ in range(NBUF)],
        ],
    )
    def lookup(tok_hbm, idx_hbm, pos_hbm, out_hbm,
               idx_v, pos_v, rows, isem, gsem, ssem):
        wid = lax.axis_index("s") * NC + lax.axis_index("c")
        seq0 = wid * cpw

        pltpu.sync_copy(pos_hbm.at[pl.ds(0, T)], pos_v)

        def idx_load_start(c, slot):
            pltpu.async_copy(
                idx_hbm.at[seq0 + c], idx_v.at[slot], isem[slot])

        def idx_load_wait(slot):
            pltpu.make_async_copy(
                idx_hbm.at[0], idx_v.at[slot], isem[slot]).wait()

        def gather_start(c, buf, slot):
            pltpu.async_copy(
                tok_hbm.at[idx_v.at[slot, 0, pl.ds(0, S0)]],
                rows[buf].at[pl.ds(0, S0)], gsem[buf])
            pltpu.async_copy(
                tok_hbm.at[idx_v.at[slot, 1, pl.ds(0, s1)]],
                rows[buf].at[pl.ds(S0, s1)], gsem[buf])

        def gather_wait(buf):
            pltpu.make_async_copy(
                tok_hbm.at[idx_v.at[0, 0, pl.ds(0, S0)]],
                rows[buf].at[pl.ds(0, S0)], gsem[buf]).wait()
            pltpu.make_async_copy(
                tok_hbm.at[idx_v.at[0, 1, pl.ds(0, s1)]],
                rows[buf].at[pl.ds(S0, s1)], gsem[buf]).wait()

        def scatter_start(c, buf):
            pltpu.async_copy(
                rows[buf], out_hbm.at[pl.ds((seq0 + c) * T, T)], ssem[buf])

        def scatter_wait(buf):
            pltpu.make_async_copy(
                rows[buf], out_hbm.at[pl.ds(0, T)], ssem[buf]).wait()

        def add_pos(buf):
            def _rows(r):
                for rr in range(2):
                    for q in range(D // LANES):
                        sl = pl.ds(q * LANES, LANES)
                        rows[buf][r + rr, sl] += pos_v[r + rr, sl]
            plsc.parallel_loop(0, T, 2, unroll=2)(_rows)

        idx_load_start(0, 0)
        idx_load_start(1, 1)
        idx_load_wait(0)
        gather_start(0, 0, 0)

        def outer(c4, _):
            for par in range(NBUF):
                c = c4 * NBUF + par
                buf = par

                gather_wait(buf)

                @pl.when(c + 2 < cpw)
                def _prefetch_idx():
                    idx_load_start(c + 2, par % 2)

                @pl.when(c + 1 < cpw)
                def _start_next():
                    idx_load_wait((par + 1) % 2)
                    nbuf = (par + 1) % NBUF
                    @pl.when(c >= NBUF - 1)
                    def _drain():
                        scatter_wait(nbuf)
                    gather_start(c + 1, nbuf, (par + 1) % 2)

                add_pos(buf)
                scatter_start(c, buf)
            return 0

        lax.fori_loop(0, cpw // NBUF, outer, 0)
        for buf in range(NBUF):
            scatter_wait(buf)

    return lookup


def kernel(idx, token_emb, pos_emb):
    B, T = idx.shape
    V, D = token_emb.shape
    idx = idx.astype(jnp.int32)
    h0 = jnp.pad(idx[:, :S0], ((0, 0), (0, IDXPAD - S0)))
    h1 = jnp.pad(idx[:, S0:], ((0, 0), (0, IDXPAD - (T - S0))))
    idx3 = jnp.stack([h0, h1], axis=1)
    lookup = _make_sc_lookup(V, D, B, T)
    out = lookup(token_emb, idx3, pos_emb)
    return out.reshape(B, T, D)


# speedup vs baseline: 8.7420x; 1.0003x over previous
import functools

import jax
import jax.numpy as jnp
from jax import lax
from jax.experimental import pallas as pl
from jax.experimental.pallas import tpu as pltpu
from jax.experimental.pallas import tpu_sc as plsc

NC = 2
NS = 16
NW = NC * NS
LANES = 16
S0 = 104
IDXPAD = 128
NBUF = 4


def _make_sc_lookup(V, D, B, T):
    s1 = T - S0
    assert 0 < s1 <= IDXPAD and s1 % 8 == 0 and T % 8 == 0
    assert B % NW == 0
    cpw = B // NW
    assert cpw % NBUF == 0 and cpw >= 2 * NBUF

    mesh = plsc.VectorSubcoreMesh(core_axis_name="c", subcore_axis_name="s")

    @functools.partial(
        pl.kernel,
        out_type=jax.ShapeDtypeStruct((B * T, D), jnp.float32),
        mesh=mesh,
        scratch_types=[
            pltpu.VMEM((2, 2, IDXPAD), jnp.int32),
            pltpu.VMEM((T, D), jnp.float32),
            [pltpu.VMEM((T, D), jnp.float32) for _ in range(NBUF)],
            [pltpu.SemaphoreType.DMA for _ in range(2)],
            [pltpu.SemaphoreType.DMA for _ in range(NBUF)],
            [pltpu.SemaphoreType.DMA for _ in range(NBUF)],
        ],
    )
    def lookup(tok_hbm, idx_hbm, pos_hbm, out_hbm,
               idx_v, pos_v, rows, isem, gsem, ssem):
        wid = lax.axis_index("s") * NC + lax.axis_index("c")
        seq0 = wid * cpw

        pltpu.sync_copy(pos_hbm.at[pl.ds(0, T)], pos_v)

        def idx_load_start(c, slot):
            pltpu.async_copy(
                idx_hbm.at[seq0 + c], idx_v.at[slot], isem[slot])

        def idx_load_wait(slot):
            pltpu.make_async_copy(
                idx_hbm.at[0], idx_v.at[slot], isem[slot]).wait()

        def gather_start(c, buf, slot):
            pltpu.async_copy(
                tok_hbm.at[idx_v.at[slot, 0, pl.ds(0, S0)]],
                rows[buf].at[pl.ds(0, S0)], gsem[buf])
            pltpu.async_copy(
                tok_hbm.at[idx_v.at[slot, 1, pl.ds(0, s1)]],
                rows[buf].at[pl.ds(S0, s1)], gsem[buf])

        def gather_wait(buf):
            pltpu.make_async_copy(
                tok_hbm.at[idx_v.at[0, 0, pl.ds(0, S0)]],
                rows[buf].at[pl.ds(0, S0)], gsem[buf]).wait()
            pltpu.make_async_copy(
                tok_hbm.at[idx_v.at[0, 1, pl.ds(0, s1)]],
                rows[buf].at[pl.ds(S0, s1)], gsem[buf]).wait()

        def scatter_start(c, buf):
            pltpu.async_copy(
                rows[buf], out_hbm.at[pl.ds((seq0 + c) * T, T)], ssem[buf])

        def scatter_wait(buf):
            pltpu.make_async_copy(
                rows[buf], out_hbm.at[pl.ds(0, T)], ssem[buf]).wait()

        def add_pos(buf):
            def _rows(r):
                for rr in range(2):
                    for q in range(D // LANES):
                        sl = pl.ds(q * LANES, LANES)
                        plsc.addupdate(rows[buf].at[r + rr, sl],
                                       pos_v[r + rr, sl])
            plsc.parallel_loop(0, T, 2, unroll=2)(_rows)

        idx_load_start(0, 0)
        idx_load_start(1, 1)
        idx_load_wait(0)
        gather_start(0, 0, 0)

        def outer(c4, _):
            for par in range(NBUF):
                c = c4 * NBUF + par
                buf = par

                gather_wait(buf)

                @pl.when(c + 2 < cpw)
                def _prefetch_idx():
                    idx_load_start(c + 2, par % 2)

                @pl.when(c + 1 < cpw)
                def _start_next():
                    idx_load_wait((par + 1) % 2)
                    nbuf = (par + 1) % NBUF
                    @pl.when(c >= NBUF - 1)
                    def _drain():
                        scatter_wait(nbuf)
                    gather_start(c + 1, nbuf, (par + 1) % 2)

                add_pos(buf)
                scatter_start(c, buf)
            return 0

        lax.fori_loop(0, cpw // NBUF, outer, 0)
        for buf in range(NBUF):
            scatter_wait(buf)

    return lookup


def kernel(idx, token_emb, pos_emb):
    B, T = idx.shape
    V, D = token_emb.shape
    idx = idx.astype(jnp.int32)
    h0 = jnp.pad(idx[:, :S0], ((0, 0), (0, IDXPAD - S0)))
    h1 = jnp.pad(idx[:, S0:], ((0, 0), (0, IDXPAD - (T - S0))))
    idx3 = jnp.stack([h0, h1], axis=1)
    lookup = _make_sc_lookup(V, D, B, T)
    out = lookup(token_emb, idx3, pos_emb)
    return out.reshape(B, T, D)
